# baseline (speedup 1.0000x reference)
import functools

import jax
import jax.numpy as jnp
from jax import lax
from jax.experimental import pallas as pl
from jax.experimental.pallas import tpu as pltpu
from jax.experimental.pallas import tpu_sc as plsc

_INDEX = 8191
_B = 4
_D = 2048

_info = plsc.get_sparse_core_info()
_NC = _info.num_cores

_mesh = plsc.ScalarSubcoreMesh(axis_name="c", num_cores=1)


@functools.partial(
    pl.kernel,
    mesh=_mesh,
    out_type=jax.ShapeDtypeStruct((_B, _D), jnp.float32),
    scratch_types=[pltpu.SemaphoreType.DMA],
)
def _get_item_sc(x_hbm, out_hbm, sem):
    pltpu.make_async_copy(x_hbm.at[:, _INDEX], out_hbm, sem).start()
    pltpu.make_async_copy(x_hbm.at[:, _INDEX], out_hbm, sem).wait()


def kernel(x):
    return _get_item_sc(x)
